# Initial kernel scaffold; baseline (speedup 1.0000x reference)
import jax
import jax.numpy as jnp
from jax.experimental import pallas as pl


def kernel(x, embedding):
    raise NotImplementedError("write your pallas kernel here")


# speedup vs baseline: 2.5714x; 2.5714x over previous
import jax
import jax.numpy as jnp
from jax.experimental import pallas as pl


def _add_block(x_ref, emb_ref, o_ref):
    o_ref[...] = x_ref[...] + emb_ref[...][None, :, :]


def kernel(x, embedding):
    B, T, D = x.shape
    emb = embedding[:T]
    Tb = 256
    grid = (T // Tb,)
    return pl.pallas_call(
        _add_block,
        grid=grid,
        in_specs=[
            pl.BlockSpec((B, Tb, D), lambda i: (0, i, 0)),
            pl.BlockSpec((Tb, D), lambda i: (i, 0)),
        ],
        out_specs=pl.BlockSpec((B, Tb, D), lambda i: (0, i, 0)),
        out_shape=jax.ShapeDtypeStruct((B, T, D), x.dtype),
    )(x, emb)
